# Initial kernel scaffold; baseline (speedup 1.0000x reference)
import jax
import jax.numpy as jnp
from jax.experimental import pallas as pl


def kernel(input, kernel, alpha):
    raise NotImplementedError("write your pallas kernel here")


# speedup vs baseline: 18.3121x; 18.3121x over previous
import functools

import jax
import jax.numpy as jnp
from jax.experimental import pallas as pl
from jax.experimental.pallas import tpu as pltpu

_TOK_BLOCK = 2048


def _rvq_body(x_ref, cblk_ref, cnorm_ref, alpha_ref, out_ref, code_ref):
    x = x_ref[...]
    cblk = cblk_ref[...]
    cnorm = cnorm_ref[...]
    alpha = alpha_ref[0, 0]
    T = x.shape[0]

    dotT = jax.lax.dot_general(
        cblk, x, (((1,), (1,)), ((), ())),
        precision=jax.lax.Precision.HIGHEST,
        preferred_element_type=jnp.float32)
    score = cnorm - 2.0 * dotT

    kiota = jax.lax.broadcasted_iota(jnp.int32, (32, T), 0)
    inds = []
    ohs = []
    for h in range(2):
        s = score[h * 32:(h + 1) * 32]
        mins = jnp.min(s, axis=0, keepdims=True)
        hit = s <= mins
        ind = jnp.min(jnp.where(hit, kiota, 32), axis=0)
        ohs.append((kiota == ind[None, :]).astype(jnp.float32))
        inds.append(ind)

    oh = jnp.concatenate(ohs, axis=0)
    v = jax.lax.dot_general(
        oh, cblk, (((0,), (0,)), ((), ())),
        precision=jax.lax.Precision.HIGHEST,
        preferred_element_type=jnp.float32)
    out_ref[...] = alpha * x + (1.0 - alpha) * v
    code_ref[...] = inds[0] + 32 * inds[1]


@functools.partial(jax.jit, static_argnames=())
def kernel(input, kernel, alpha):
    B, S, D = input.shape
    n_tok = B * S
    x = input.reshape(n_tok, D)
    alpha_arr = jnp.asarray(alpha, jnp.float32).reshape(1, 1)
    cblk = jnp.zeros((64, 64), jnp.float32)
    cblk = cblk.at[:32, :32].set(kernel[0]).at[32:, 32:].set(kernel[1])
    cnorm = jnp.sum(cblk * cblk, axis=1, keepdims=True)
    grid = (n_tok // _TOK_BLOCK,)
    out, code = pl.pallas_call(
        _rvq_body,
        grid=grid,
        in_specs=[
            pl.BlockSpec((_TOK_BLOCK, D), lambda i: (i, 0)),
            pl.BlockSpec((64, 64), lambda i: (0, 0)),
            pl.BlockSpec((64, 1), lambda i: (0, 0)),
            pl.BlockSpec(memory_space=pltpu.SMEM),
        ],
        out_specs=[
            pl.BlockSpec((_TOK_BLOCK, D), lambda i: (i, 0)),
            pl.BlockSpec((_TOK_BLOCK,), lambda i: (i,)),
        ],
        out_shape=[
            jax.ShapeDtypeStruct((n_tok, D), jnp.float32),
            jax.ShapeDtypeStruct((n_tok,), jnp.int32),
        ],
    )(x, cblk, cnorm, alpha_arr)
    return out.reshape(B, S, D), code.reshape(B, S)


# speedup vs baseline: 52.6789x; 2.8767x over previous
import functools

import jax
import jax.numpy as jnp
from jax.experimental import pallas as pl
from jax.experimental.pallas import tpu as pltpu

_B_BLK = 8


def _rvq_body(x_ref, cs_ref, chi_ref, clo_ref, cnorm_ref, alpha_ref,
              out_ref, code_ref):
    cs = cs_ref[...]
    chi = chi_ref[...]
    clo = clo_ref[...]
    cnorm = cnorm_ref[...]
    alpha = alpha_ref[0, 0]
    S = x_ref.shape[2]
    kiota = jax.lax.broadcasted_iota(jnp.int32, (32, S), 0)

    for b in range(_B_BLK):
        x = x_ref[b]

        dotT = jax.lax.dot_general(
            cs, x, (((1,), (0,)), ((), ())),
            precision=jax.lax.Precision.HIGHEST,
            preferred_element_type=jnp.float32)
        score = cnorm + dotT

        inds = []
        ohs = []
        for h in range(2):
            s = score[h * 32:(h + 1) * 32]
            mins = jnp.min(s, axis=0, keepdims=True)
            ind = jnp.min(jnp.where(s <= mins, kiota, 32), axis=0)
            ohs.append((kiota == ind[None, :]).astype(jnp.bfloat16))
            inds.append(ind)

        oh = jnp.concatenate(ohs, axis=0)
        v = jax.lax.dot_general(
            chi, oh, (((0,), (0,)), ((), ())),
            preferred_element_type=jnp.float32)
        v = v + jax.lax.dot_general(
            clo, oh, (((0,), (0,)), ((), ())),
            preferred_element_type=jnp.float32)
        out_ref[b] = alpha * x + v
        code_ref[pl.ds(b * S, S)] = inds[0] + 32 * inds[1]


@functools.partial(jax.jit, static_argnames=())
def kernel(input, kernel, alpha):
    B, S, D = input.shape
    xt = jnp.transpose(input, (0, 2, 1))
    alpha_f = jnp.asarray(alpha, jnp.float32)
    alpha_arr = alpha_f.reshape(1, 1)
    cblk = jnp.zeros((2 * 32, D), jnp.float32)
    cblk = cblk.at[:32, :32].set(kernel[0]).at[32:, 32:].set(kernel[1])
    cnorm = jnp.sum(cblk * cblk, axis=1, keepdims=True)
    cs = -2.0 * cblk
    c2 = (1.0 - alpha_f) * cblk
    chi = c2.astype(jnp.bfloat16)
    clo = (c2 - chi.astype(jnp.float32)).astype(jnp.bfloat16)
    grid = (B // _B_BLK,)
    out_t, code = pl.pallas_call(
        _rvq_body,
        grid=grid,
        in_specs=[
            pl.BlockSpec((_B_BLK, D, S), lambda i: (i, 0, 0)),
            pl.BlockSpec((64, D), lambda i: (0, 0)),
            pl.BlockSpec((64, D), lambda i: (0, 0)),
            pl.BlockSpec((64, D), lambda i: (0, 0)),
            pl.BlockSpec((64, 1), lambda i: (0, 0)),
            pl.BlockSpec(memory_space=pltpu.SMEM),
        ],
        out_specs=[
            pl.BlockSpec((_B_BLK, D, S), lambda i: (i, 0, 0)),
            pl.BlockSpec((_B_BLK * S,), lambda i: (i,)),
        ],
        out_shape=[
            jax.ShapeDtypeStruct((B, D, S), jnp.float32),
            jax.ShapeDtypeStruct((B * S,), jnp.int32),
        ],
    )(xt, cs, chi, clo, cnorm, alpha_arr)
    return jnp.transpose(out_t, (0, 2, 1)), code.reshape(B, S)
